# Initial kernel scaffold; baseline (speedup 1.0000x reference)
import jax
import jax.numpy as jnp
from jax.experimental import pallas as pl


def kernel(x):
    raise NotImplementedError("write your pallas kernel here")


# speedup vs baseline: 45.6750x; 45.6750x over previous
import numpy as np
import jax
import jax.numpy as jnp
from jax.experimental import pallas as pl
from jax.experimental.pallas import tpu as pltpu

_WINDOW = 96
_SENT = 120
_FS = 500
_DOWN_FS = 100
_NPATCH = _SENT // 12
_MAXPEAKS = 25


def _downsample_weights(L=5000, scale=_DOWN_FS / _FS):
    Lo = int(L * scale)
    src = (np.arange(Lo) + 0.5) / scale - 0.5
    src = np.clip(src, 0.0, L - 1)
    lo = np.floor(src).astype(np.int64)
    hi = np.minimum(lo + 1, L - 1)
    frac = (src - lo).astype(np.float32)
    base = 5 * np.arange(Lo)
    olo = lo - base - 1
    ohi = hi - base - 1
    assert olo.min() >= 0 and ohi.max() <= 2, "static interp offsets changed"
    w = np.zeros((3, Lo), np.float32)
    w[olo, np.arange(Lo)] += (1.0 - frac)
    w[ohi, np.arange(Lo)] += frac
    return w


_W3 = _downsample_weights()
_LD = _W3.shape[1]


def _qrs_body(xt_ref, w_ref, patch_ref, t_ref, s_ref,
              xds_s, idx_s, jm_s):
    B = xt_ref.shape[0]
    Ld = _LD
    W = _WINDOW
    P = _NPATCH
    S = _SENT

    xt = xt_ref[...]
    w = w_ref[...]
    xds = (xt[:, 0] * jnp.reshape(w[0], (1, 1, Ld))
           + xt[:, 1] * jnp.reshape(w[1], (1, 1, Ld))
           + xt[:, 2] * jnp.reshape(w[2], (1, 1, Ld)))

    sig = xds[:, 0, :]
    mu = jnp.mean(sig, axis=1, keepdims=True)
    var = jnp.mean((sig - mu) ** 2, axis=1, keepdims=True)
    thr = mu + 1.5 * jnp.sqrt(var)
    sl = jnp.concatenate([sig[:, :1], sig[:, :-1]], axis=1)
    sr = jnp.concatenate([sig[:, 1:], sig[:, -1:]], axis=1)
    ii = jax.lax.broadcasted_iota(jnp.int32, (B, Ld), 1)
    cand = ((sig > sl) & (sig >= sr) & (sig > thr)
            & (ii >= 1) & (ii <= Ld - 2))

    amp = jnp.where(cand, sig, -jnp.inf)
    kept = jnp.zeros((B, Ld), dtype=jnp.bool_)
    for _ in range(_MAXPEAKS):
        rowmax = jnp.max(amp, axis=1, keepdims=True)
        valid = rowmax > -jnp.inf
        pm = jnp.min(jnp.where(amp == rowmax, ii, Ld), axis=1, keepdims=True)
        kept = kept | (valid & (ii == pm))
        amp = jnp.where(valid & (jnp.abs(ii - pm) < 40), -jnp.inf, amp)

    has = jnp.any(kept, axis=1, keepdims=True)
    smax = jnp.max(sig, axis=1, keepdims=True)
    pmax = jnp.min(jnp.where(sig == smax, ii, Ld), axis=1, keepdims=True)
    kept = (kept & has) | ((ii == pmax) & jnp.logical_not(has))
    m = jnp.sum(kept.astype(jnp.int32), axis=1, keepdims=True)

    cols = []
    kk = kept
    for _ in range(_MAXPEAKS):
        ps = jnp.min(jnp.where(kk, ii, Ld), axis=1, keepdims=True)
        cols.append(jnp.where(ps < Ld, ps, 0))
        kk = kk & (ii != ps)
    qrs = jnp.concatenate(cols, axis=1)

    pp = jax.lax.broadcasted_iota(jnp.int32, (B, W), 1)
    idx_cols = []
    for j in range(P):
        qj = qrs[:, j:j + 1]
        qn = qrs[:, j + 1:j + 2]
        if j == 0:
            left = jnp.zeros_like(qj)
        else:
            qp = qrs[:, j - 1:j]
            left = (qp + qj) // 2
        right = jnp.where(m == j + 1, Ld, (qj + qn) // 2)
        right = jnp.minimum(right, Ld)
        ws = jnp.maximum(right - left, 1)
        off = jnp.where(ws < W, (W - ws) // 2, -((ws - W) // 2))
        idx_j = left + jnp.clip(pp - off, 0, ws - 1)
        idx_cols.append(jnp.clip(idx_j, 0, Ld - 1))
    idx_all = jnp.concatenate(idx_cols, axis=1)

    jm = (jax.lax.broadcasted_iota(jnp.int32, (B, P * W), 1) // W) < m
    xds_s[...] = xds
    idx_s[...] = idx_all
    jm_s[...] = jm.astype(jnp.float32)
    iot_l = jax.lax.broadcasted_iota(jnp.int32, (Ld, P * W), 0)

    def gbody(b, carry):
        xb = xds_s[pl.ds(b, 1)][0]
        idxb = idx_s[pl.ds(b, 1)]
        mb = jm_s[pl.ds(b, 1)]
        oh = (iot_l == idxb).astype(jnp.float32)
        pb = jnp.dot(xb, oh, preferred_element_type=jnp.float32)
        pb = pb * mb
        patch_ref[pl.ds(b, 1)] = pb[None]
        return carry

    jax.lax.fori_loop(0, B, gbody, 0)

    ps2 = jax.lax.broadcasted_iota(jnp.int32, (B, S), 1)
    vals = qrs // 100 + 1
    pmod = ps2 % m
    t = jnp.zeros((B, S), jnp.int32)
    for s in range(_MAXPEAKS):
        t = t + jnp.where(pmod == s, vals[:, s:s + 1], 0)
    act = ps2 < 12 * m
    t_ref[...] = jnp.where(act, t, 0)
    s_ref[...] = jnp.where(act, ps2 // m + 1, 0)


def kernel(x):
    B, C, L = x.shape
    Ld = _LD
    xr = jnp.reshape(x, (B, C, Ld, 5))
    xt = jnp.transpose(xr[..., 1:4], (0, 3, 1, 2))
    w = jnp.asarray(_W3)

    patch, t, s = pl.pallas_call(
        _qrs_body,
        out_shape=[
            jax.ShapeDtypeStruct((B, 12, _NPATCH * _WINDOW), jnp.float32),
            jax.ShapeDtypeStruct((B, _SENT), jnp.int32),
            jax.ShapeDtypeStruct((B, _SENT), jnp.int32),
        ],
        scratch_shapes=[
            pltpu.VMEM((B, 12, Ld), jnp.float32),
            pltpu.VMEM((B, _NPATCH * _WINDOW), jnp.int32),
            pltpu.VMEM((B, _NPATCH * _WINDOW), jnp.float32),
        ],
    )(xt, w)

    patches = jnp.transpose(
        jnp.reshape(patch, (B, 12, _NPATCH, _WINDOW)), (0, 2, 1, 3))
    x_pad = jnp.reshape(patches, (B, _SENT, _WINDOW))
    x_pad = jnp.reshape(x_pad, (B, 12, -1, _WINDOW))
    return (x_pad, t, s)
